# baseline (speedup 1.0000x reference)
import functools

import jax
import jax.numpy as jnp
from jax import lax
from jax.experimental import pallas as pl
from jax.experimental.pallas import tpu as pltpu
from jax.experimental.pallas import tpu_sc as plsc

CODEBOOK_SIZE = 8192
CODEBOOK_DIM = 32
_N_CHUNK = 128
_V_SUB = 512


_V_CHUNK = 4096


def _dist_argmin_body(z_ref, e_ref, tok_ref, sc_ref):
    z = z_ref[0]
    C = z.shape[0]
    n_j = z.shape[1] // _N_CHUNK
    n_k = CODEBOOK_SIZE // _V_CHUNK
    n_vb = _V_CHUNK // _V_SUB
    for j in range(n_j):
        zc = z[:, j * _N_CHUNK:(j + 1) * _N_CHUNK]
        zsq = zc * zc
        z_norm = zsq[0:1, :]
        for k in range(1, C):
            z_norm = z_norm + zsq[k:k + 1, :]
        z_bf = zc.astype(jnp.bfloat16).astype(jnp.float32)
        acc_v = jnp.full((1, _N_CHUNK), jnp.inf, jnp.float32)
        acc_i = jnp.zeros((1, _N_CHUNK), jnp.int32)
        for k in range(n_k):
            def vb_step(vb, carry, k=k):
                mv, mi = carry
                base = k * _V_CHUNK + vb * _V_SUB
                eb_raw = e_ref[pl.ds(base, _V_SUB), :]
                eb = eb_raw.astype(jnp.bfloat16).astype(jnp.float32)
                enb = jnp.sum(eb_raw * eb_raw, axis=1, keepdims=True)
                sc = eb[:, 0:1] * z_bf[0:1, :]
                for c in range(1, C):
                    sc = sc + eb[:, c:c + 1] * z_bf[c:c + 1, :]
                dvb = (z_norm + enb) - 2.0 * sc
                mv_b = jnp.min(dvb, axis=0, keepdims=True)
                rows = lax.broadcasted_iota(jnp.int32, dvb.shape, 0)
                cand = jnp.where(dvb == mv_b, rows, jnp.int32(CODEBOOK_SIZE))
                mi_b = jnp.min(cand, axis=0, keepdims=True) + base
                tb = mv_b < mv
                return jnp.where(tb, mv_b, mv), jnp.where(tb, mi_b, mi)

            mv, mi = lax.fori_loop(
                0, n_vb, vb_step,
                (jnp.full((1, _N_CHUNK), jnp.inf, jnp.float32),
                 jnp.zeros((1, _N_CHUNK), jnp.int32)))
            take = mv < acc_v
            acc_i = jnp.where(take, mi, acc_i)
            acc_v = jnp.where(take, mv, acc_v).astype(jnp.bfloat16).astype(jnp.float32)
        row, col = divmod(j * _N_CHUNK, 128)
        tok_ref[0, row, col:col + _N_CHUNK] = acc_i[0]


def _tc_tokens(z3, embedding):
    B, C, HW = z3.shape
    n_row = HW // 128
    out = pl.pallas_call(
        _dist_argmin_body,
        grid=(B,),
        in_specs=[
            pl.BlockSpec((1, C, HW), lambda b: (b, 0, 0)),
            pl.BlockSpec((CODEBOOK_SIZE, C), lambda b: (0, 0)),
        ],
        out_specs=pl.BlockSpec((1, n_row, 128), lambda b: (b, 0, 0)),
        out_shape=jax.ShapeDtypeStruct((B, n_row, 128), jnp.int32),
        scratch_shapes=[pltpu.VMEM((_V_SUB, _N_CHUNK), jnp.float32)],
    )(z3, embedding)
    return out


try:
    _SC_INFO = plsc.get_sparse_core_info()
    _NC = _SC_INFO.num_cores
    _NS = _SC_INFO.num_subcores
except Exception:
    _NC, _NS = 2, 16
_NW = _NC * _NS
_IDX_PER_STREAM = 128


_GATHER_DIM = 128


@functools.lru_cache(maxsize=None)
def _make_sc_gather(n_tok, dim):
    per_w = n_tok // _NW
    n_chunk = per_w // _IDX_PER_STREAM
    mesh = plsc.VectorSubcoreMesh(core_axis_name="c", subcore_axis_name="s")

    @functools.partial(
        pl.kernel,
        mesh=mesh,
        out_type=jax.ShapeDtypeStruct((n_tok, dim), jnp.float32),
        scratch_types=[
            pltpu.VMEM((n_chunk, _IDX_PER_STREAM), jnp.int32),
            pltpu.VMEM((per_w, dim), jnp.float32),
            pltpu.SemaphoreType.DMA,
        ],
    )
    def gather(table_hbm, idx_hbm, out_hbm, idx_v, rows_v, sem):
        wid = lax.axis_index("s") * _NC + lax.axis_index("c")
        pltpu.sync_copy(idx_hbm.at[pl.ds(wid * n_chunk, n_chunk)], idx_v)
        copies = []
        for g in range(n_chunk):
            copies.append(pltpu.async_copy(
                table_hbm.at[idx_v.at[g]],
                rows_v.at[pl.ds(g * _IDX_PER_STREAM, _IDX_PER_STREAM)],
                sem,
            ))
        for c in copies:
            c.wait()
        pltpu.sync_copy(rows_v, out_hbm.at[pl.ds(wid * per_w, per_w)])

    return gather


def kernel(z_enc, embedding):
    B, C, H, W = z_enc.shape
    z3 = z_enc.reshape(B, C, H * W)
    token3 = _tc_tokens(z3, embedding)
    n_tok = B * H * W
    idx2d = token3.reshape(n_tok // _IDX_PER_STREAM, _IDX_PER_STREAM)
    emb_pad = jnp.pad(embedding, ((0, 0), (0, _GATHER_DIM - C)))
    rows = _make_sc_gather(n_tok, _GATHER_DIM)(emb_pad, idx2d)
    z_q = rows[:, :C].reshape(B, H, W, C).transpose(0, 3, 1, 2)
    return z_q, token3.reshape(B, H * W)
